# Initial kernel scaffold; baseline (speedup 1.0000x reference)
import jax
import jax.numpy as jnp
from jax.experimental import pallas as pl


def kernel(x, edge_index, W, b):
    raise NotImplementedError("write your pallas kernel here")


# speedup vs baseline: 9.2648x; 9.2648x over previous
import jax
import jax.numpy as jnp
from jax import lax
from jax.experimental import pallas as pl
from jax.experimental.pallas import tpu as pltpu
from jax.experimental.pallas import tpu_sc as plsc

N_NODES = 10000
IN_DIM = 256
HID_DIM = 512
NC = 2
NS = 16
L = 16
HALF = IN_DIM // NC
CHUNK = 128
ZROWS = 632
ACC_ROWS = NS * ZROWS
BM = 1000


def _mesh():
    return plsc.VectorSubcoreMesh(
        core_axis_name="c", subcore_axis_name="s",
        num_cores=NC, num_subcores=NS)


def _deg_body(dst_hbm, ones_hbm, zeros_hbm, out_hbm, didx_v, ones_v, acc_sh):
    c = lax.axis_index("c")
    s = lax.axis_index("s")
    pltpu.sync_copy(zeros_hbm, acc_sh.at[pl.ds(s * ZROWS, ZROWS)])
    pltpu.sync_copy(ones_hbm, ones_v)
    plsc.subcore_barrier()
    edges_per_tile = dst_hbm.shape[0] // (NC * NS)
    base = (c * NS + s) * edges_per_tile

    def body(i, carry):
        pltpu.sync_copy(dst_hbm.at[pl.ds(base + i * CHUNK, CHUNK)], didx_v)
        pltpu.sync_copy(ones_v, acc_sh.at[didx_v], add=True)
        return carry

    lax.fori_loop(0, edges_per_tile // CHUNK, body, 0)
    plsc.subcore_barrier()
    pltpu.sync_copy(
        acc_sh.at[pl.ds(s * ZROWS, ZROWS)],
        out_hbm.at[pl.ds(c * ACC_ROWS + s * ZROWS, ZROWS)])


def _deg_call(dst_p, ones_hbm, zeros_hbm):
    return pl.kernel(
        _deg_body,
        out_type=jax.ShapeDtypeStruct((NC * ACC_ROWS, HALF), jnp.float32),
        mesh=_mesh(),
        scratch_types=[
            pltpu.VMEM((CHUNK,), jnp.int32),
            pltpu.VMEM((CHUNK, HALF), jnp.float32),
            pltpu.MemorySpace.VMEM_SHARED((ACC_ROWS, HALF), jnp.float32),
        ],
    )(dst_p, ones_hbm, zeros_hbm)


def _agg_body(src_hbm, dst_hbm, xs_hbm, zeros_hbm, out_hbm,
              sidx_v, didx_v, rows_v, acc_sh, sem):
    c = lax.axis_index("c")
    s = lax.axis_index("s")
    pltpu.sync_copy(zeros_hbm, acc_sh.at[pl.ds(s * ZROWS, ZROWS)])
    plsc.subcore_barrier()
    edges_per_tile = src_hbm.shape[0] // NS
    base = s * edges_per_tile
    off = jnp.full((L,), c * N_NODES, jnp.int32)

    def body(i, carry):
        eb = base + i * CHUNK
        pltpu.sync_copy(src_hbm.at[pl.ds(eb, CHUNK)], sidx_v)
        pltpu.sync_copy(dst_hbm.at[pl.ds(eb, CHUNK)], didx_v)
        for j in range(CHUNK // L):
            sl = pl.ds(j * L, L)
            sidx_v[sl] = sidx_v[sl] + off
        pltpu.async_copy(xs_hbm.at[sidx_v], rows_v, sem).wait()
        pltpu.sync_copy(rows_v, acc_sh.at[didx_v], add=True)
        return carry

    lax.fori_loop(0, edges_per_tile // CHUNK, body, 0)
    plsc.subcore_barrier()
    pltpu.sync_copy(
        acc_sh.at[pl.ds(s * ZROWS, ZROWS)],
        out_hbm.at[pl.ds(c * ACC_ROWS + s * ZROWS, ZROWS)])


def _agg_call(src_p, dst_p, xs_flat, zeros_hbm):
    return pl.kernel(
        _agg_body,
        out_type=jax.ShapeDtypeStruct((NC * ACC_ROWS, HALF), jnp.float32),
        mesh=_mesh(),
        scratch_types=[
            pltpu.VMEM((CHUNK,), jnp.int32),
            pltpu.VMEM((CHUNK,), jnp.int32),
            pltpu.VMEM((CHUNK, HALF), jnp.float32),
            pltpu.MemorySpace.VMEM_SHARED((ACC_ROWS, HALF), jnp.float32),
            pltpu.SemaphoreType.DMA,
        ],
    )(src_p, dst_p, xs_flat, zeros_hbm)


def _xs_body(dp_ref, x_ref, xs_ref):
    deg = dp_ref[0, :, 0:1] + dp_ref[1, :, 0:1] + 1.0
    xs_ref[...] = lax.rsqrt(deg) * x_ref[...]


def _xs_call(deg3, x):
    nb = N_NODES // BM
    return pl.pallas_call(
        _xs_body,
        grid=(NC, nb),
        in_specs=[
            pl.BlockSpec((NC, BM, L), lambda c, i: (0, i, 0)),
            pl.BlockSpec((BM, HALF), lambda c, i: (i, c)),
        ],
        out_specs=pl.BlockSpec((BM, HALF), lambda c, i: (c * nb + i, 0)),
        out_shape=jax.ShapeDtypeStruct((NC * N_NODES, HALF), jnp.float32),
    )(deg3, x)


def _mm_body(dp_ref, s_ref, xs_ref, w_ref, b_ref, o_ref):
    dinv = lax.rsqrt(dp_ref[0, :, 0:1] + dp_ref[1, :, 0:1] + 1.0)
    acc = b_ref[...] + jnp.dot(
        dinv * (s_ref[0] + xs_ref[0]), w_ref[0],
        preferred_element_type=jnp.float32, precision=lax.Precision.HIGHEST)
    acc = acc + jnp.dot(
        dinv * (s_ref[1] + xs_ref[1]), w_ref[1],
        preferred_element_type=jnp.float32, precision=lax.Precision.HIGHEST)
    o_ref[...] = jnp.maximum(acc, 0.0)


def _mm_call(deg3, s3, xs3, w3, b2):
    return pl.pallas_call(
        _mm_body,
        grid=(N_NODES // BM,),
        in_specs=[
            pl.BlockSpec((NC, BM, L), lambda i: (0, i, 0)),
            pl.BlockSpec((NC, BM, HALF), lambda i: (0, i, 0)),
            pl.BlockSpec((NC, BM, HALF), lambda i: (0, i, 0)),
            pl.BlockSpec((NC, HALF, HID_DIM), lambda i: (0, 0, 0)),
            pl.BlockSpec((1, HID_DIM), lambda i: (0, 0)),
        ],
        out_specs=pl.BlockSpec((BM, HID_DIM), lambda i: (i, 0)),
        out_shape=jax.ShapeDtypeStruct((N_NODES, HID_DIM), jnp.float32),
    )(deg3, s3, xs3, w3, b2)


def kernel(x, edge_index, W, b):
    e = edge_index.shape[1]
    quant = NC * NS * CHUNK
    e_pad = ((e + quant - 1) // quant) * quant
    pad = e_pad - e
    src_p = jnp.concatenate(
        [edge_index[0], jnp.zeros((pad,), jnp.int32)])
    dst_p = jnp.concatenate(
        [edge_index[1], jnp.full((pad,), N_NODES, jnp.int32)])

    ones_rows = jnp.ones((CHUNK, HALF), jnp.float32)
    zeros_agg = jnp.zeros((ZROWS, HALF), jnp.float32)

    deg_flat = _deg_call(dst_p, ones_rows, zeros_agg)
    deg3 = deg_flat.reshape(NC, ACC_ROWS, HALF)[:, :N_NODES, :L]
    xs_flat = _xs_call(deg3, x)
    s_flat = _agg_call(src_p, dst_p, xs_flat, zeros_agg)

    s3 = s_flat.reshape(NC, ACC_ROWS, HALF)[:, :N_NODES]
    xs3 = xs_flat.reshape(NC, N_NODES, HALF)
    w3 = W.reshape(NC, HALF, HID_DIM)
    b2 = b.reshape(1, HID_DIM)
    return _mm_call(deg3, s3, xs3, w3, b2)


# speedup vs baseline: 12.2515x; 1.3224x over previous
import jax
import jax.numpy as jnp
from jax import lax
from jax.experimental import pallas as pl
from jax.experimental.pallas import tpu as pltpu
from jax.experimental.pallas import tpu_sc as plsc

N_NODES = 10000
IN_DIM = 256
HID_DIM = 512
NC = 2
NS = 16
L = 16
HALF = IN_DIM // NC
CHUNK = 128
ZROWS = 632
ACC_ROWS = NS * ZROWS
BM = 1000


def _mesh():
    return plsc.VectorSubcoreMesh(
        core_axis_name="c", subcore_axis_name="s",
        num_cores=NC, num_subcores=NS)


def _deg_body(dst_hbm, ones_hbm, zeros_hbm, out_hbm, didx_v, ones_v, acc_sh):
    c = lax.axis_index("c")
    s = lax.axis_index("s")
    pltpu.sync_copy(zeros_hbm, acc_sh.at[pl.ds(s * ZROWS, ZROWS)])
    pltpu.sync_copy(ones_hbm, ones_v)
    plsc.subcore_barrier()
    edges_per_tile = dst_hbm.shape[0] // (NC * NS)
    base = (c * NS + s) * edges_per_tile

    def body(i, carry):
        pltpu.sync_copy(dst_hbm.at[pl.ds(base + i * CHUNK, CHUNK)], didx_v)
        pltpu.sync_copy(ones_v, acc_sh.at[didx_v], add=True)
        return carry

    lax.fori_loop(0, edges_per_tile // CHUNK, body, 0)
    plsc.subcore_barrier()
    pltpu.sync_copy(
        acc_sh.at[pl.ds(s * ZROWS, ZROWS)],
        out_hbm.at[pl.ds(c * ACC_ROWS + s * ZROWS, ZROWS)])


def _deg_call(dst_p, ones_hbm, zeros_hbm):
    return pl.kernel(
        _deg_body,
        out_type=jax.ShapeDtypeStruct((NC * ACC_ROWS, HALF), jnp.float32),
        mesh=_mesh(),
        scratch_types=[
            pltpu.VMEM((CHUNK,), jnp.int32),
            pltpu.VMEM((CHUNK, HALF), jnp.float32),
            pltpu.MemorySpace.VMEM_SHARED((ACC_ROWS, HALF), jnp.float32),
        ],
    )(dst_p, ones_hbm, zeros_hbm)


NBUF = 2


def _agg_body(src_hbm, dst_hbm, xs_hbm, zeros_hbm, out_hbm,
              sidx_v, didx_v, rows_v, acc_sh,
              is0, is1, gs0, gs1):
    isem = (is0, is1)
    gsem = (gs0, gs1)
    c = lax.axis_index("c")
    s = lax.axis_index("s")
    pltpu.sync_copy(zeros_hbm, acc_sh.at[pl.ds(s * ZROWS, ZROWS)])
    plsc.subcore_barrier()
    edges_per_tile = src_hbm.shape[0] // NS
    nch = edges_per_tile // CHUNK
    base = s * edges_per_tile
    off = jnp.full((L,), c * N_NODES, jnp.int32)

    def fire_idx(i, b):
        eb = base + i * CHUNK
        pltpu.async_copy(src_hbm.at[pl.ds(eb, CHUNK)], sidx_v.at[b], isem[b])
        pltpu.async_copy(dst_hbm.at[pl.ds(eb, CHUNK)], didx_v.at[b], isem[b])

    def wait_idx(b):
        pltpu.make_async_copy(src_hbm.at[pl.ds(0, CHUNK)], sidx_v.at[b], isem[b]).wait()
        pltpu.make_async_copy(src_hbm.at[pl.ds(0, CHUNK)], didx_v.at[b], isem[b]).wait()

    def fire_gather(b):
        for j in range(CHUNK // L):
            sl = pl.ds(j * L, L)
            sidx_v[b, sl] = sidx_v[b, sl] + off
        pltpu.async_copy(xs_hbm.at[sidx_v.at[b]], rows_v.at[b], gsem[b])

    def wait_gather(b):
        pltpu.make_async_copy(xs_hbm.at[pl.ds(0, CHUNK)], rows_v.at[b], gsem[b]).wait()

    fire_idx(0, 0)
    fire_idx(1, 1)
    wait_idx(0)
    fire_gather(0)

    def outer(o, carry):
        for b in range(NBUF):
            i = o * NBUF + b
            b1 = (b + 1) % NBUF

            @pl.when(i + 1 < nch)
            def _():
                wait_idx(b1)
                fire_gather(b1)

            wait_gather(b)
            pltpu.sync_copy(rows_v.at[b], acc_sh.at[didx_v.at[b]], add=True)

            @pl.when(i + NBUF < nch)
            def _():
                fire_idx(i + NBUF, b)
        return carry

    lax.fori_loop(0, nch // NBUF, outer, 0)
    plsc.subcore_barrier()
    pltpu.sync_copy(
        acc_sh.at[pl.ds(s * ZROWS, ZROWS)],
        out_hbm.at[pl.ds(c * ACC_ROWS + s * ZROWS, ZROWS)])


def _agg_call(src_p, dst_p, xs_flat, zeros_hbm):
    return pl.kernel(
        _agg_body,
        out_type=jax.ShapeDtypeStruct((NC * ACC_ROWS, HALF), jnp.float32),
        mesh=_mesh(),
        scratch_types=[
            pltpu.VMEM((NBUF, CHUNK), jnp.int32),
            pltpu.VMEM((NBUF, CHUNK), jnp.int32),
            pltpu.VMEM((NBUF, CHUNK, HALF), jnp.float32),
            pltpu.MemorySpace.VMEM_SHARED((ACC_ROWS, HALF), jnp.float32),
        ] + [pltpu.SemaphoreType.DMA] * 4,
    )(src_p, dst_p, xs_flat, zeros_hbm)


def _xs_body(dp_ref, x_ref, xs_ref):
    deg = dp_ref[0, :, 0:1] + dp_ref[1, :, 0:1] + 1.0
    xs_ref[...] = lax.rsqrt(deg) * x_ref[...]


def _xs_call(deg3, x):
    nb = N_NODES // BM
    return pl.pallas_call(
        _xs_body,
        grid=(NC, nb),
        in_specs=[
            pl.BlockSpec((NC, BM, L), lambda c, i: (0, i, 0)),
            pl.BlockSpec((BM, HALF), lambda c, i: (i, c)),
        ],
        out_specs=pl.BlockSpec((BM, HALF), lambda c, i: (c * nb + i, 0)),
        out_shape=jax.ShapeDtypeStruct((NC * N_NODES, HALF), jnp.float32),
    )(deg3, x)


def _mm_body(dp_ref, s_ref, xs_ref, w_ref, b_ref, o_ref):
    dinv = lax.rsqrt(dp_ref[0, :, 0:1] + dp_ref[1, :, 0:1] + 1.0)
    acc = b_ref[...] + jnp.dot(
        dinv * (s_ref[0] + xs_ref[0]), w_ref[0],
        preferred_element_type=jnp.float32)
    acc = acc + jnp.dot(
        dinv * (s_ref[1] + xs_ref[1]), w_ref[1],
        preferred_element_type=jnp.float32)
    o_ref[...] = jnp.maximum(acc, 0.0)


def _mm_call(deg3, s3, xs3, w3, b2):
    return pl.pallas_call(
        _mm_body,
        grid=(N_NODES // BM,),
        in_specs=[
            pl.BlockSpec((NC, BM, L), lambda i: (0, i, 0)),
            pl.BlockSpec((NC, BM, HALF), lambda i: (0, i, 0)),
            pl.BlockSpec((NC, BM, HALF), lambda i: (0, i, 0)),
            pl.BlockSpec((NC, HALF, HID_DIM), lambda i: (0, 0, 0)),
            pl.BlockSpec((1, HID_DIM), lambda i: (0, 0)),
        ],
        out_specs=pl.BlockSpec((BM, HID_DIM), lambda i: (i, 0)),
        out_shape=jax.ShapeDtypeStruct((N_NODES, HID_DIM), jnp.float32),
    )(deg3, s3, xs3, w3, b2)


def kernel(x, edge_index, W, b):
    e = edge_index.shape[1]
    quant = NC * NS * CHUNK
    e_pad = ((e + quant - 1) // quant) * quant
    pad = e_pad - e
    src_p = jnp.concatenate(
        [edge_index[0], jnp.zeros((pad,), jnp.int32)])
    dst_p = jnp.concatenate(
        [edge_index[1], jnp.full((pad,), N_NODES, jnp.int32)])

    ones_rows = jnp.ones((CHUNK, HALF), jnp.float32)
    zeros_agg = jnp.zeros((ZROWS, HALF), jnp.float32)

    deg_flat = _deg_call(dst_p, ones_rows, zeros_agg)
    deg3 = deg_flat.reshape(NC, ACC_ROWS, HALF)[:, :N_NODES, :L]
    xs_flat = _xs_call(deg3, x)
    s_flat = _agg_call(src_p, dst_p, xs_flat, zeros_agg)

    s3 = s_flat.reshape(NC, ACC_ROWS, HALF)
    xs3 = xs_flat.reshape(NC, N_NODES, HALF)
    w3 = W.reshape(NC, HALF, HID_DIM)
    b2 = b.reshape(1, HID_DIM)
    return _mm_call(deg3, s3, xs3, w3, b2)
